# baseline (speedup 1.0000x reference)
import functools
import math

import jax
import jax.numpy as jnp
from jax import lax
from jax.experimental import pallas as pl
from jax.experimental.pallas import tpu as pltpu
from jax.experimental.pallas import tpu_sc as plsc

NUM_NODES = 1000000
NUM_FEAT = 4
N_SAMPLED = 100
D = 64
B = 16384
SP = 128
LOG_NN1 = math.log(float(NUM_NODES) + 1.0)

_SC_PARAMS = dict(
    compiler_params=pltpu.CompilerParams(
        needs_layout_passes=False, use_tc_tiling_on_sc=False))


def _sc_mesh():
    return plsc.VectorSubcoreMesh(core_axis_name="c", subcore_axis_name="s")


def _wid(NC):
    return lax.axis_index("s") * NC + lax.axis_index("c")


def _sc_a_body(NC, CH,
               si, bi2, bl2, ncw, ncb, sidx,
               bfo, two, tbo, swo, sbo,
               idxb, lblb, featb, twb, tbb, sxb, swb, sbb, sem):
    wid = _wid(NC)

    @pl.when(wid == 0)
    def _():
        pltpu.sync_copy(sidx, sxb)
        cw = pltpu.async_copy(ncw.at[sxb], swb, sem)
        cb = pltpu.async_copy(ncb.at[sxb], sbb, sem)
        cw.wait()
        cb.wait()
        pltpu.sync_copy(swb, swo)
        pltpu.sync_copy(sbb, sbo)

    for j in range(CH):
        pltpu.sync_copy(bi2.at[wid * CH + j], idxb[j])
        pltpu.sync_copy(bl2.at[wid * CH + j], lblb[j])

    for j in range(CH):
        c0 = pltpu.async_copy(si.at[idxb[j]], featb, sem)
        c1 = pltpu.async_copy(ncw.at[lblb[j]], twb, sem)
        c2 = pltpu.async_copy(ncb.at[lblb[j]], tbb, sem)
        c0.wait(); c1.wait(); c2.wait()
        base = wid * (CH * 128) + j * 128
        pltpu.sync_copy(featb, bfo.at[pl.ds(base, 128)])
        pltpu.sync_copy(twb, two.at[pl.ds(base, 128)])
        pltpu.sync_copy(tbb, tbo.at[pl.ds(base, 128)])


def _sc_gather_a(si, bi2, bl2, ncw, ncb, sidx):
    info = plsc.get_sparse_core_info()
    NC, NS = info.num_cores, info.num_subcores
    CH = B // (NC * NS * 128)
    f32, i32 = jnp.float32, jnp.int32
    out_type = (
        jax.ShapeDtypeStruct((B, 8), i32),
        jax.ShapeDtypeStruct((B, D), f32),
        jax.ShapeDtypeStruct((B,), f32),
        jax.ShapeDtypeStruct((SP, D), f32),
        jax.ShapeDtypeStruct((SP,), f32),
    )
    scratch = [
        [pltpu.VMEM((128,), i32) for _ in range(CH)],
        [pltpu.VMEM((128,), i32) for _ in range(CH)],
        pltpu.VMEM((128, 8), i32),
        pltpu.VMEM((128, D), f32),
        pltpu.VMEM((128,), f32),
        pltpu.VMEM((SP,), i32),
        pltpu.VMEM((SP, D), f32),
        pltpu.VMEM((SP,), f32),
        pltpu.SemaphoreType.DMA,
    ]
    body = functools.partial(_sc_a_body, NC, CH)
    return pl.kernel(body, out_type=out_type, mesh=_sc_mesh(),
                     scratch_types=scratch, **_SC_PARAMS)(
        si, bi2, bl2, ncw, ncb, sidx)


def _sc_b_body(NC, CH,
               f0, f1, f2, f3, e0t, e1t, e2t, e3t, alp,
               e0o, e1o, e2o, e3o, ao,
               fb, e0b, e1b, e2b, e3b, ab, sem):
    wid = _wid(NC)
    for j in range(CH):
        row = wid * CH + j
        pltpu.sync_copy(f0.at[row], fb[0])
        pltpu.sync_copy(f1.at[row], fb[1])
        pltpu.sync_copy(f2.at[row], fb[2])
        pltpu.sync_copy(f3.at[row], fb[3])
        c0 = pltpu.async_copy(e0t.at[fb[0]], e0b, sem)
        c1 = pltpu.async_copy(e1t.at[fb[1]], e1b, sem)
        c2 = pltpu.async_copy(e2t.at[fb[2]], e2b, sem)
        c3 = pltpu.async_copy(e3t.at[fb[3]], e3b, sem)
        ca = pltpu.async_copy(alp.at[fb[0]], ab, sem)
        c0.wait(); c1.wait(); c2.wait(); c3.wait(); ca.wait()
        base = wid * (CH * 128) + j * 128
        pltpu.sync_copy(e0b, e0o.at[pl.ds(base, 128)])
        pltpu.sync_copy(e1b, e1o.at[pl.ds(base, 128)])
        pltpu.sync_copy(e2b, e2o.at[pl.ds(base, 128)])
        pltpu.sync_copy(e3b, e3o.at[pl.ds(base, 128)])
        pltpu.sync_copy(ab, ao.at[pl.ds(base, 128)])


def _sc_gather_b(f0, f1, f2, f3, e0t, e1t, e2t, e3t, alp):
    info = plsc.get_sparse_core_info()
    NC, NS = info.num_cores, info.num_subcores
    CH = B // (NC * NS * 128)
    f32, i32 = jnp.float32, jnp.int32
    out_type = (
        jax.ShapeDtypeStruct((B, D), f32),
        jax.ShapeDtypeStruct((B, D), f32),
        jax.ShapeDtypeStruct((B, D), f32),
        jax.ShapeDtypeStruct((B, D), f32),
        jax.ShapeDtypeStruct((B, 8), f32),
    )
    scratch = [
        [pltpu.VMEM((128,), i32) for _ in range(NUM_FEAT)],
        pltpu.VMEM((128, D), f32),
        pltpu.VMEM((128, D), f32),
        pltpu.VMEM((128, D), f32),
        pltpu.VMEM((128, D), f32),
        pltpu.VMEM((128, 8), f32),
        pltpu.SemaphoreType.DMA,
    ]
    body = functools.partial(_sc_b_body, NC, CH)
    return pl.kernel(body, out_type=out_type, mesh=_sc_mesh(),
                     scratch_types=scratch, **_SC_PARAMS)(
        f0, f1, f2, f3, e0t, e1t, e2t, e3t, alp)


_BS = 1024
_G = B // _BS


def _tc_body(e0, e1, e2, e3, tw, a, tb, lbl, sw, sb2, sids, coff, out):
    i = pl.program_id(0)
    ae = jnp.exp(a[...][:, :NUM_FEAT])
    denom = jnp.sum(ae, axis=1, keepdims=True)
    merge = (e0[...] * ae[:, 0:1] + e1[...] * ae[:, 1:2]
             + e2[...] * ae[:, 2:3] + e3[...] * ae[:, 3:4]) / denom

    lblf = lbl[...].astype(jnp.float32)
    tprob = (jnp.log(lblf + 2.0) - jnp.log(lblf + 1.0)) / LOG_NN1
    t = (jnp.sum(merge * tw[...], axis=1, keepdims=True) + tb[...]
         - jnp.log(tprob * float(N_SAMPLED)))

    s = lax.dot_general(merge, sw[...], (((1,), (1,)), ((), ())),
                        preferred_element_type=jnp.float32)
    s = s + sb2[...] + coff[...]
    hit = sids[...] == lbl[...]
    s = jnp.where(hit, jnp.float32(-1e9), s)

    m = jnp.maximum(jnp.max(s, axis=1, keepdims=True), t)
    p = jnp.sum(jnp.exp(s - m), axis=1, keepdims=True) + jnp.exp(t - m)
    bsum = jnp.sum(jnp.log(p) + m - t)

    acc = jnp.where(i == 0, bsum, out[...][0, 0] + bsum)
    out[...] = jnp.where(i == _G - 1, acc / B, acc).reshape(1, 1)


def _tc_loss(e0, e1, e2, e3, tw, a, tb, lbl, sw, sb2, sids, coff):
    bs = _BS
    row = lambda i: (i, 0)
    fix = lambda i: (0, 0)
    specs = [
        pl.BlockSpec((bs, D), row),
        pl.BlockSpec((bs, D), row),
        pl.BlockSpec((bs, D), row),
        pl.BlockSpec((bs, D), row),
        pl.BlockSpec((bs, D), row),
        pl.BlockSpec((bs, 8), row),
        pl.BlockSpec((bs, 1), row),
        pl.BlockSpec((bs, 1), row),
        pl.BlockSpec((SP, D), fix),
        pl.BlockSpec((1, SP), fix),
        pl.BlockSpec((1, SP), fix),
        pl.BlockSpec((1, SP), fix),
    ]
    return pl.pallas_call(
        _tc_body,
        grid=(_G,),
        in_specs=specs,
        out_specs=pl.BlockSpec((1, 1), fix),
        out_shape=jax.ShapeDtypeStruct((1, 1), jnp.float32),
        compiler_params=pltpu.CompilerParams(
            dimension_semantics=("arbitrary",)),
    )(e0, e1, e2, e3, tw, a, tb, lbl, sw, sb2, sids, coff)


def kernel(side_info, batch_index, batch_labels, nce_w, nce_b,
           emb0, emb1, emb2, emb3, alpha_embedding):
    i32, f32 = jnp.int32, jnp.float32
    si = jnp.pad(side_info.astype(i32), ((0, 0), (0, 8 - NUM_FEAT)))
    bi = batch_index.astype(i32)
    bl = batch_labels.astype(i32)
    bi2 = bi.reshape(B // 128, 128)
    bl2 = bl.reshape(B // 128, 128)

    u = jax.random.uniform(jax.random.key(42), (N_SAMPLED,), dtype=f32)
    sampled = jnp.floor(jnp.exp(u * LOG_NN1)).astype(i32) - 1
    sampled = jnp.clip(sampled, 0, NUM_NODES - 1)
    pad = SP - N_SAMPLED
    sidx = jnp.concatenate([sampled, jnp.zeros((pad,), i32)])
    sids = jnp.concatenate([sampled, jnp.full((pad,), -1, i32)]).reshape(1, SP)
    sf = sampled.astype(f32)
    sprob = (jnp.log(sf + 2.0) - jnp.log(sf + 1.0)) / LOG_NN1
    coff = jnp.concatenate([
        -jnp.log(sprob * float(N_SAMPLED)),
        jnp.full((pad,), -1e9, f32),
    ]).reshape(1, SP)

    alp = jnp.pad(alpha_embedding, ((0, 0), (0, 8 - NUM_FEAT)))
    bf, twr, tbr, swr, sbr = _sc_gather_a(si, bi2, bl2, nce_w, nce_b, sidx)

    fcols = [bf[:, i].reshape(B // 128, 128) for i in range(NUM_FEAT)]
    e0r, e1r, e2r, e3r, ar = _sc_gather_b(
        fcols[0], fcols[1], fcols[2], fcols[3],
        emb0, emb1, emb2, emb3, alp)

    out = _tc_loss(e0r, e1r, e2r, e3r, twr, ar, tbr.reshape(B, 1),
                   bl.reshape(B, 1), swr, sbr.reshape(1, SP), sids, coff)
    return out[0, 0]


# speedup vs baseline: 1.0947x; 1.0947x over previous
import functools
import math

import jax
import jax.numpy as jnp
from jax import lax
from jax.experimental import pallas as pl
from jax.experimental.pallas import tpu as pltpu
from jax.experimental.pallas import tpu_sc as plsc

NUM_NODES = 1000000
NUM_FEAT = 4
N_SAMPLED = 100
D = 64
B = 16384
SP = 128
LOG_NN1 = math.log(float(NUM_NODES) + 1.0)

_SC_PARAMS = dict(
    compiler_params=pltpu.CompilerParams(
        needs_layout_passes=False, use_tc_tiling_on_sc=False))


def _sc_mesh():
    return plsc.VectorSubcoreMesh(core_axis_name="c", subcore_axis_name="s")


def _wid(NC):
    return lax.axis_index("s") * NC + lax.axis_index("c")


_PB = 2000


def _pack_body(a, b, out):
    out[...] = jnp.concatenate([a[...], b[...]], axis=1)


def _pack_halves(x):
    n = x.shape[0]
    h = n // 2
    g = h // _PB
    return pl.pallas_call(
        _pack_body,
        grid=(g,),
        in_specs=[
            pl.BlockSpec((_PB, D), lambda i: (i, 0)),
            pl.BlockSpec((_PB, D), lambda i, hb=h // _PB: (i + hb, 0)),
        ],
        out_specs=pl.BlockSpec((_PB, 2 * D), lambda i: (i, 0)),
        out_shape=jax.ShapeDtypeStruct((h, 2 * D), jnp.float32),
        compiler_params=pltpu.CompilerParams(
            dimension_semantics=("arbitrary",)),
    )(x, x)


def _sc_a_body(NC, CH,
               si, bi2, bl2m, bl2, ncw, ncb, sidxm, sidx,
               bfo, two, tbo, swo, sbo,
               idxb, lblmb, lblb, featb, twb, tbb, sxmb, sxb, swb, sbb, sem):
    wid = _wid(NC)

    @pl.when(wid == 0)
    def _():
        pltpu.sync_copy(sidxm, sxmb)
        pltpu.sync_copy(sidx, sxb)
        cw = pltpu.async_copy(ncw.at[sxmb], swb, sem)
        cb = pltpu.async_copy(ncb.at[sxb], sbb, sem)
        cw.wait()
        cb.wait()
        pltpu.sync_copy(swb, swo)
        pltpu.sync_copy(sbb, sbo)

    for j in range(CH):
        pltpu.sync_copy(bi2.at[wid * CH + j], idxb[j])
        pltpu.sync_copy(bl2m.at[wid * CH + j], lblmb[j])
        pltpu.sync_copy(bl2.at[wid * CH + j], lblb[j])

    for j in range(CH):
        c0 = pltpu.async_copy(si.at[idxb[j]], featb, sem)
        c1 = pltpu.async_copy(ncw.at[lblmb[j]], twb, sem)
        c2 = pltpu.async_copy(ncb.at[lblb[j]], tbb, sem)
        c0.wait(); c1.wait(); c2.wait()
        base = wid * (CH * 128) + j * 128
        pltpu.sync_copy(featb, bfo.at[pl.ds(base, 128)])
        pltpu.sync_copy(twb, two.at[pl.ds(base, 128)])
        pltpu.sync_copy(tbb, tbo.at[pl.ds(base, 128)])


def _sc_gather_a(si, bi2, bl2m, bl2, ncw, ncb, sidxm, sidx):
    info = plsc.get_sparse_core_info()
    NC, NS = info.num_cores, info.num_subcores
    CH = B // (NC * NS * 128)
    f32, i32 = jnp.float32, jnp.int32
    out_type = (
        jax.ShapeDtypeStruct((B, 8), i32),
        jax.ShapeDtypeStruct((B, 2 * D), f32),
        jax.ShapeDtypeStruct((B,), f32),
        jax.ShapeDtypeStruct((SP, 2 * D), f32),
        jax.ShapeDtypeStruct((SP,), f32),
    )
    scratch = [
        [pltpu.VMEM((128,), i32) for _ in range(CH)],
        [pltpu.VMEM((128,), i32) for _ in range(CH)],
        [pltpu.VMEM((128,), i32) for _ in range(CH)],
        pltpu.VMEM((128, 8), i32),
        pltpu.VMEM((128, 2 * D), f32),
        pltpu.VMEM((128,), f32),
        pltpu.VMEM((SP,), i32),
        pltpu.VMEM((SP,), i32),
        pltpu.VMEM((SP, 2 * D), f32),
        pltpu.VMEM((SP,), f32),
        pltpu.SemaphoreType.DMA,
    ]
    body = functools.partial(_sc_a_body, NC, CH)
    return pl.kernel(body, out_type=out_type, mesh=_sc_mesh(),
                     scratch_types=scratch, **_SC_PARAMS)(
        si, bi2, bl2m, bl2, ncw, ncb, sidxm, sidx)


def _sc_b_body(NC, CH,
               f0, f1, f2, f3, fa, e0t, e1t, e2t, e3t, alp,
               e0o, e1o, e2o, e3o, ao,
               fb, e0b, e1b, e2b, e3b, ab, sem):
    wid = _wid(NC)
    for j in range(CH):
        row = wid * CH + j
        pltpu.sync_copy(f0.at[row], fb[0])
        pltpu.sync_copy(f1.at[row], fb[1])
        pltpu.sync_copy(f2.at[row], fb[2])
        pltpu.sync_copy(f3.at[row], fb[3])
        pltpu.sync_copy(fa.at[row], fb[4])
        c0 = pltpu.async_copy(e0t.at[fb[0]], e0b, sem)
        c1 = pltpu.async_copy(e1t.at[fb[1]], e1b, sem)
        c2 = pltpu.async_copy(e2t.at[fb[2]], e2b, sem)
        c3 = pltpu.async_copy(e3t.at[fb[3]], e3b, sem)
        ca = pltpu.async_copy(alp.at[fb[4]], ab, sem)
        c0.wait(); c1.wait(); c2.wait(); c3.wait(); ca.wait()
        base = wid * (CH * 128) + j * 128
        pltpu.sync_copy(e0b, e0o.at[pl.ds(base, 128)])
        pltpu.sync_copy(e1b, e1o.at[pl.ds(base, 128)])
        pltpu.sync_copy(e2b, e2o.at[pl.ds(base, 128)])
        pltpu.sync_copy(e3b, e3o.at[pl.ds(base, 128)])
        pltpu.sync_copy(ab, ao.at[pl.ds(base, 128)])


def _sc_gather_b(f0, f1, f2, f3, fa, e0t, e1t, e2t, e3t, alp):
    info = plsc.get_sparse_core_info()
    NC, NS = info.num_cores, info.num_subcores
    CH = B // (NC * NS * 128)
    f32, i32 = jnp.float32, jnp.int32
    out_type = (
        jax.ShapeDtypeStruct((B, 2 * D), f32),
        jax.ShapeDtypeStruct((B, D), f32),
        jax.ShapeDtypeStruct((B, D), f32),
        jax.ShapeDtypeStruct((B, D), f32),
        jax.ShapeDtypeStruct((B, 8), f32),
    )
    scratch = [
        [pltpu.VMEM((128,), i32) for _ in range(NUM_FEAT + 1)],
        pltpu.VMEM((128, 2 * D), f32),
        pltpu.VMEM((128, D), f32),
        pltpu.VMEM((128, D), f32),
        pltpu.VMEM((128, D), f32),
        pltpu.VMEM((128, 8), f32),
        pltpu.SemaphoreType.DMA,
    ]
    body = functools.partial(_sc_b_body, NC, CH)
    return pl.kernel(body, out_type=out_type, mesh=_sc_mesh(),
                     scratch_types=scratch, **_SC_PARAMS)(
        f0, f1, f2, f3, fa, e0t, e1t, e2t, e3t, alp)


_BS = 1024
_G = B // _BS


def _tc_body(e0p, h0, e1, e2, e3, twp, hl, a, tb, lbl, sw, sb2, sids, coff,
             out):
    i = pl.program_id(0)
    e0f = e0p[...]
    e0 = jnp.where(h0[...] > 0, e0f[:, D:], e0f[:, :D])
    twf = twp[...]
    tw = jnp.where(hl[...] > 0, twf[:, D:], twf[:, :D])
    ae = jnp.exp(a[...][:, :NUM_FEAT])
    denom = jnp.sum(ae, axis=1, keepdims=True)
    merge = (e0 * ae[:, 0:1] + e1[...] * ae[:, 1:2]
             + e2[...] * ae[:, 2:3] + e3[...] * ae[:, 3:4]) / denom

    lblf = lbl[...].astype(jnp.float32)
    tprob = (jnp.log(lblf + 2.0) - jnp.log(lblf + 1.0)) / LOG_NN1
    t = (jnp.sum(merge * tw, axis=1, keepdims=True) + tb[...]
         - jnp.log(tprob * float(N_SAMPLED)))

    s = lax.dot_general(merge, sw[...], (((1,), (1,)), ((), ())),
                        preferred_element_type=jnp.float32)
    s = s + sb2[...] + coff[...]
    hit = sids[...] == lbl[...]
    s = jnp.where(hit, jnp.float32(-1e9), s)

    m = jnp.maximum(jnp.max(s, axis=1, keepdims=True), t)
    p = jnp.sum(jnp.exp(s - m), axis=1, keepdims=True) + jnp.exp(t - m)
    bsum = jnp.sum(jnp.log(p) + m - t)

    acc = jnp.where(i == 0, bsum, out[...][0, 0] + bsum)
    out[...] = jnp.where(i == _G - 1, acc / B, acc).reshape(1, 1)


def _tc_loss(e0p, h0, e1, e2, e3, twp, hl, a, tb, lbl, sw, sb2, sids, coff):
    bs = _BS
    row = lambda i: (i, 0)
    fix = lambda i: (0, 0)
    specs = [
        pl.BlockSpec((bs, 2 * D), row),
        pl.BlockSpec((bs, 1), row),
        pl.BlockSpec((bs, D), row),
        pl.BlockSpec((bs, D), row),
        pl.BlockSpec((bs, D), row),
        pl.BlockSpec((bs, 2 * D), row),
        pl.BlockSpec((bs, 1), row),
        pl.BlockSpec((bs, 8), row),
        pl.BlockSpec((bs, 1), row),
        pl.BlockSpec((bs, 1), row),
        pl.BlockSpec((SP, D), fix),
        pl.BlockSpec((1, SP), fix),
        pl.BlockSpec((1, SP), fix),
        pl.BlockSpec((1, SP), fix),
    ]
    return pl.pallas_call(
        _tc_body,
        grid=(_G,),
        in_specs=specs,
        out_specs=pl.BlockSpec((1, 1), fix),
        out_shape=jax.ShapeDtypeStruct((1, 1), jnp.float32),
        compiler_params=pltpu.CompilerParams(
            dimension_semantics=("arbitrary",)),
    )(e0p, h0, e1, e2, e3, twp, hl, a, tb, lbl, sw, sb2, sids, coff)


def kernel(side_info, batch_index, batch_labels, nce_w, nce_b,
           emb0, emb1, emb2, emb3, alpha_embedding):
    i32, f32 = jnp.int32, jnp.float32
    si = jnp.pad(side_info.astype(i32), ((0, 0), (0, 8 - NUM_FEAT)))
    bi = batch_index.astype(i32)
    bl = batch_labels.astype(i32)
    bi2 = bi.reshape(B // 128, 128)
    bl2 = bl.reshape(B // 128, 128)

    u = jax.random.uniform(jax.random.key(42), (N_SAMPLED,), dtype=f32)
    sampled = jnp.floor(jnp.exp(u * LOG_NN1)).astype(i32) - 1
    sampled = jnp.clip(sampled, 0, NUM_NODES - 1)
    pad = SP - N_SAMPLED
    sidx = jnp.concatenate([sampled, jnp.zeros((pad,), i32)])
    sids = jnp.concatenate([sampled, jnp.full((pad,), -1, i32)]).reshape(1, SP)
    sf = sampled.astype(f32)
    sprob = (jnp.log(sf + 2.0) - jnp.log(sf + 1.0)) / LOG_NN1
    coff = jnp.concatenate([
        -jnp.log(sprob * float(N_SAMPLED)),
        jnp.full((pad,), -1e9, f32),
    ]).reshape(1, SP)

    alp = jnp.pad(alpha_embedding, ((0, 0), (0, 8 - NUM_FEAT)))
    H = NUM_NODES // 2
    ncwp = _pack_halves(nce_w)
    e0tp = _pack_halves(emb0)

    bl2m = (bl % H).reshape(B // 128, 128)
    sidxm = sidx % H
    sh = (sidx // H).reshape(1, SP)
    bf, twr, tbr, swr, sbr = _sc_gather_a(
        si, bi2, bl2m, bl2, ncwp, nce_b, sidxm, sidx)
    swsel = jnp.where(sh.reshape(SP, 1) > 0, swr[:, D:], swr[:, :D])

    f0 = bf[:, 0]
    fcols = [(f0 % H).reshape(B // 128, 128)] + [
        bf[:, i].reshape(B // 128, 128) for i in range(1, NUM_FEAT)] + [
        f0.reshape(B // 128, 128)]
    e0r, e1r, e2r, e3r, ar = _sc_gather_b(
        fcols[0], fcols[1], fcols[2], fcols[3], fcols[4],
        e0tp, emb1, emb2, emb3, alp)

    h0 = (f0 // H).astype(i32).reshape(B, 1)
    hl = (bl // H).astype(i32).reshape(B, 1)
    out = _tc_loss(e0r, h0, e1r, e2r, e3r, twr, hl, ar, tbr.reshape(B, 1),
                   bl.reshape(B, 1), swsel, sbr.reshape(1, SP), sids, coff)
    return out[0, 0]


# speedup vs baseline: 2.0995x; 1.9178x over previous
import functools
import math

import jax
import jax.numpy as jnp
from jax import lax
from jax.experimental import pallas as pl
from jax.experimental.pallas import tpu as pltpu
from jax.experimental.pallas import tpu_sc as plsc

NUM_NODES = 1000000
NUM_FEAT = 4
N_SAMPLED = 100
D = 64
B = 16384
SP = 128
LOG_NN1 = math.log(float(NUM_NODES) + 1.0)

_SC_PARAMS = dict(
    compiler_params=pltpu.CompilerParams(
        needs_layout_passes=False, use_tc_tiling_on_sc=False))


def _sc_mesh():
    return plsc.VectorSubcoreMesh(core_axis_name="c", subcore_axis_name="s")


def _wid(NC):
    return lax.axis_index("s") * NC + lax.axis_index("c")


_PB = 2000


def _pack_body(a, b, out):
    out[...] = jnp.concatenate([a[...], b[...]], axis=1)


def _pack_halves(x):
    n = x.shape[0]
    h = n // 2
    g = h // _PB
    return pl.pallas_call(
        _pack_body,
        grid=(g,),
        in_specs=[
            pl.BlockSpec((_PB, D), lambda i: (i, 0)),
            pl.BlockSpec((_PB, D), lambda i, hb=h // _PB: (i + hb, 0)),
        ],
        out_specs=pl.BlockSpec((_PB, 2 * D), lambda i: (i, 0)),
        out_shape=jax.ShapeDtypeStruct((h, 2 * D), jnp.float32),
        compiler_params=pltpu.CompilerParams(
            dimension_semantics=("arbitrary",)),
    )(x, x)


def _sc_a_body(NC, CH,
               si, bi2, bl2m, bl2, ncw, ncb, sidxm, sidx,
               bfo, two, tbo, swo, sbo,
               idxb, lblmb, lblb, featb, twb, tbb, sxmb, sxb, swb, sbb, sem):
    wid = _wid(NC)

    @pl.when(wid == 0)
    def _():
        pltpu.sync_copy(sidxm, sxmb)
        pltpu.sync_copy(sidx, sxb)
        cw = pltpu.async_copy(ncw.at[sxmb], swb, sem)
        cb = pltpu.async_copy(ncb.at[sxb], sbb, sem)
        cw.wait()
        cb.wait()
        pltpu.sync_copy(swb, swo)
        pltpu.sync_copy(sbb, sbo)

    for j in range(CH):
        pltpu.sync_copy(bi2.at[wid * CH + j], idxb[j])
        pltpu.sync_copy(bl2m.at[wid * CH + j], lblmb[j])
        pltpu.sync_copy(bl2.at[wid * CH + j], lblb[j])

    for j in range(CH):
        c0 = pltpu.async_copy(si.at[idxb[j]], featb, sem)
        c1 = pltpu.async_copy(ncw.at[lblmb[j]], twb, sem)
        c2 = pltpu.async_copy(ncb.at[lblb[j]], tbb, sem)
        c0.wait(); c1.wait(); c2.wait()
        base = wid * (CH * 128) + j * 128
        pltpu.sync_copy(featb, bfo.at[pl.ds(base, 128)])
        pltpu.sync_copy(twb, two.at[pl.ds(base, 128)])
        pltpu.sync_copy(tbb, tbo.at[pl.ds(base, 128)])


def _sc_gather_a(si, bi2, bl2m, bl2, ncw, ncb, sidxm, sidx):
    info = plsc.get_sparse_core_info()
    NC, NS = info.num_cores, info.num_subcores
    CH = B // (NC * NS * 128)
    f32, i32 = jnp.float32, jnp.int32
    out_type = (
        jax.ShapeDtypeStruct((B, 8), i32),
        jax.ShapeDtypeStruct((B, 2 * D), f32),
        jax.ShapeDtypeStruct((B,), f32),
        jax.ShapeDtypeStruct((SP, 2 * D), f32),
        jax.ShapeDtypeStruct((SP,), f32),
    )
    scratch = [
        [pltpu.VMEM((128,), i32) for _ in range(CH)],
        [pltpu.VMEM((128,), i32) for _ in range(CH)],
        [pltpu.VMEM((128,), i32) for _ in range(CH)],
        pltpu.VMEM((128, 8), i32),
        pltpu.VMEM((128, 2 * D), f32),
        pltpu.VMEM((128,), f32),
        pltpu.VMEM((SP,), i32),
        pltpu.VMEM((SP,), i32),
        pltpu.VMEM((SP, 2 * D), f32),
        pltpu.VMEM((SP,), f32),
        pltpu.SemaphoreType.DMA,
    ]
    body = functools.partial(_sc_a_body, NC, CH)
    return pl.kernel(body, out_type=out_type, mesh=_sc_mesh(),
                     scratch_types=scratch, **_SC_PARAMS)(
        si, bi2, bl2m, bl2, ncw, ncb, sidxm, sidx)


def _sc_b_body(NC, CH,
               f0, f1, f2, f3, fa, e0t, e1t, e2t, e3t, alp,
               e0o, e1o, e2o, e3o, ao,
               fb, e0b, e1b, e2b, e3b, ab, sem):
    wid = _wid(NC)
    for j in range(CH):
        row = wid * CH + j
        pltpu.sync_copy(f0.at[row], fb[0])
        pltpu.sync_copy(f1.at[row], fb[1])
        pltpu.sync_copy(f2.at[row], fb[2])
        pltpu.sync_copy(f3.at[row], fb[3])
        pltpu.sync_copy(fa.at[row], fb[4])
        c0 = pltpu.async_copy(e0t.at[fb[0]], e0b, sem)
        c1 = pltpu.async_copy(e1t.at[fb[1]], e1b, sem)
        c2 = pltpu.async_copy(e2t.at[fb[2]], e2b, sem)
        c3 = pltpu.async_copy(e3t.at[fb[3]], e3b, sem)
        ca = pltpu.async_copy(alp.at[fb[4]], ab, sem)
        c0.wait(); c1.wait(); c2.wait(); c3.wait(); ca.wait()
        base = wid * (CH * 128) + j * 128
        pltpu.sync_copy(e0b, e0o.at[pl.ds(base, 128)])
        pltpu.sync_copy(e1b, e1o.at[pl.ds(base, 128)])
        pltpu.sync_copy(e2b, e2o.at[pl.ds(base, 128)])
        pltpu.sync_copy(e3b, e3o.at[pl.ds(base, 128)])
        pltpu.sync_copy(ab, ao.at[pl.ds(base, 128)])


def _sc_gather_b(f0, f1, f2, f3, fa, e0t, e1t, e2t, e3t, alp):
    info = plsc.get_sparse_core_info()
    NC, NS = info.num_cores, info.num_subcores
    CH = B // (NC * NS * 128)
    f32, i32 = jnp.float32, jnp.int32
    out_type = (
        jax.ShapeDtypeStruct((B, 2 * D), f32),
        jax.ShapeDtypeStruct((B, D), f32),
        jax.ShapeDtypeStruct((B, D), f32),
        jax.ShapeDtypeStruct((B, D), f32),
        jax.ShapeDtypeStruct((B, 8), i32),
    )
    scratch = [
        [pltpu.VMEM((128,), i32) for _ in range(NUM_FEAT + 1)],
        pltpu.VMEM((128, 2 * D), f32),
        pltpu.VMEM((128, D), f32),
        pltpu.VMEM((128, D), f32),
        pltpu.VMEM((128, D), f32),
        pltpu.VMEM((128, 8), i32),
        pltpu.SemaphoreType.DMA,
    ]
    body = functools.partial(_sc_b_body, NC, CH)
    return pl.kernel(body, out_type=out_type, mesh=_sc_mesh(),
                     scratch_types=scratch, **_SC_PARAMS)(
        f0, f1, f2, f3, fa, e0t, e1t, e2t, e3t, alp)


_BS = 1024
_G = B // _BS


def _tc_body(e0p, h0, e1, e2, e3, twp, hl, a, tb, lbl, sw, sb2, sids, coff,
             out):
    i = pl.program_id(0)
    e0f = e0p[...]
    e0 = jnp.where(h0[...] > 0, e0f[:, D:], e0f[:, :D])
    twf = twp[...]
    tw = jnp.where(hl[...] > 0, twf[:, D:], twf[:, :D])
    ae = jnp.exp(a[...])
    denom = jnp.sum(ae, axis=1, keepdims=True)
    merge = (e0 * ae[:, 0:1] + e1[...] * ae[:, 1:2]
             + e2[...] * ae[:, 2:3] + e3[...] * ae[:, 3:4]) / denom

    lblf = lbl[...].astype(jnp.float32)
    tprob = (jnp.log(lblf + 2.0) - jnp.log(lblf + 1.0)) / LOG_NN1
    t = (jnp.sum(merge * tw, axis=1, keepdims=True) + tb[...]
         - jnp.log(tprob * float(N_SAMPLED)))

    s = lax.dot_general(merge, sw[...], (((1,), (1,)), ((), ())),
                        preferred_element_type=jnp.float32)
    s = s + sb2[...] + coff[...]
    hit = sids[...] == lbl[...]
    s = jnp.where(hit, jnp.float32(-1e9), s)

    m = jnp.maximum(jnp.max(s, axis=1, keepdims=True), t)
    p = jnp.sum(jnp.exp(s - m), axis=1, keepdims=True) + jnp.exp(t - m)
    bsum = jnp.sum(jnp.log(p) + m - t)

    acc = jnp.where(i == 0, bsum, out[...][0, 0] + bsum)
    out[...] = jnp.where(i == _G - 1, acc / B, acc).reshape(1, 1)


def _tc_loss(e0p, h0, e1, e2, e3, twp, hl, a, tb, lbl, sw, sb2, sids, coff):
    bs = _BS
    row = lambda i: (i, 0)
    fix = lambda i: (0, 0)
    specs = [
        pl.BlockSpec((bs, 2 * D), row),
        pl.BlockSpec((bs, 1), row),
        pl.BlockSpec((bs, D), row),
        pl.BlockSpec((bs, D), row),
        pl.BlockSpec((bs, D), row),
        pl.BlockSpec((bs, 2 * D), row),
        pl.BlockSpec((bs, 1), row),
        pl.BlockSpec((bs, NUM_FEAT), row),
        pl.BlockSpec((bs, 1), row),
        pl.BlockSpec((bs, 1), row),
        pl.BlockSpec((SP, D), fix),
        pl.BlockSpec((1, SP), fix),
        pl.BlockSpec((1, SP), fix),
        pl.BlockSpec((1, SP), fix),
    ]
    return pl.pallas_call(
        _tc_body,
        grid=(_G,),
        in_specs=specs,
        out_specs=pl.BlockSpec((1, 1), fix),
        out_shape=jax.ShapeDtypeStruct((1, 1), jnp.float32),
        compiler_params=pltpu.CompilerParams(
            dimension_semantics=("arbitrary",)),
    )(e0p, h0, e1, e2, e3, twp, hl, a, tb, lbl, sw, sb2, sids, coff)


def kernel(side_info, batch_index, batch_labels, nce_w, nce_b,
           emb0, emb1, emb2, emb3, alpha_embedding):
    i32, f32 = jnp.int32, jnp.float32
    si = jnp.concatenate(
        [side_info.astype(i32),
         lax.bitcast_convert_type(alpha_embedding, i32)], axis=1)
    bi = batch_index.astype(i32)
    bl = batch_labels.astype(i32)
    bi2 = bi.reshape(B // 128, 128)
    bl2 = bl.reshape(B // 128, 128)

    u = jax.random.uniform(jax.random.key(42), (N_SAMPLED,), dtype=f32)
    sampled = jnp.floor(jnp.exp(u * LOG_NN1)).astype(i32) - 1
    sampled = jnp.clip(sampled, 0, NUM_NODES - 1)
    pad = SP - N_SAMPLED
    sidx = jnp.concatenate([sampled, jnp.zeros((pad,), i32)])
    sids = jnp.concatenate([sampled, jnp.full((pad,), -1, i32)]).reshape(1, SP)
    sf = sampled.astype(f32)
    sprob = (jnp.log(sf + 2.0) - jnp.log(sf + 1.0)) / LOG_NN1
    coff = jnp.concatenate([
        -jnp.log(sprob * float(N_SAMPLED)),
        jnp.full((pad,), -1e9, f32),
    ]).reshape(1, SP)

    H = NUM_NODES // 2
    ncwp = _pack_halves(nce_w)
    e0tp = _pack_halves(emb0)

    bl2m = (bl % H).reshape(B // 128, 128)
    sidxm = sidx % H
    sh = (sidx // H).reshape(1, SP)
    bf, twr, tbr, swr, sbr = _sc_gather_a(
        si, bi2, bl2m, bl2, ncwp, nce_b, sidxm, sidx)
    swsel = jnp.where(sh.reshape(SP, 1) > 0, swr[:, D:], swr[:, :D])

    f0 = bf[:, 0]
    fcols = [(f0 % H).reshape(B // 128, 128)] + [
        bf[:, i].reshape(B // 128, 128) for i in range(1, NUM_FEAT)] + [
        f0.reshape(B // 128, 128)]
    e0r, e1r, e2r, e3r, ar = _sc_gather_b(
        fcols[0], fcols[1], fcols[2], fcols[3], fcols[4],
        e0tp, emb1, emb2, emb3, si)
    af = lax.bitcast_convert_type(ar[:, NUM_FEAT:], jnp.float32)

    h0 = (f0 // H).astype(i32).reshape(B, 1)
    hl = (bl // H).astype(i32).reshape(B, 1)
    out = _tc_loss(e0r, h0, e1r, e2r, e3r, twr, hl, af, tbr.reshape(B, 1),
                   bl.reshape(B, 1), swsel, sbr.reshape(1, SP), sids, coff)
    return out[0, 0]
